# Initial kernel scaffold; baseline (speedup 1.0000x reference)
import jax
import jax.numpy as jnp
from jax.experimental import pallas as pl


def kernel(x, params):
    raise NotImplementedError("write your pallas kernel here")


# speedup vs baseline: 1.3674x; 1.3674x over previous
import functools

import jax
import jax.numpy as jnp
from jax import lax
from jax.experimental import pallas as pl
from jax.experimental.pallas import tpu as pltpu

B = 64
IMG = 224
PS = 14
NPATCH = (IMG // PS) ** 2
PD = PS * PS
D = 128
NE = 16
NH = 8
DH = D // NH
HID = 256
L = B
T = NPATCH * B

NB = 16
TCH = 1024


def _embed_kernel(xp_ref, wt_ref, b_ref, pos_ref, o_ref):
    xp = xp_ref[...]
    y = xp.reshape(NB * B, PD) @ wt_ref[...] + b_ref[...]
    y = y.reshape(NB, B, D) + pos_ref[...].reshape(NB, 1, D)
    o_ref[...] = y


def _attn_kernel(x_ref, ipt_ref, ipb_ref, wc_ref, bc_ref, wot_ref, ob_ref,
                 o_ref, *, nb):
    x = x_ref[...].reshape(nb * L, D)
    xi = x @ ipt_ref[...] + ipb_ref[...]
    qkv = xi @ wc_ref[...] + bc_ref[...]
    hh = lax.broadcasted_iota(jnp.int32, (NH, 1, D), 0)
    dd = lax.broadcasted_iota(jnp.int32, (NH, 1, D), 2)
    msk = (dd // DH == hh).astype(jnp.float32)
    scale = 1.0 / (float(DH) ** 0.5)

    outs = []
    for i in range(nb):
        q = qkv[i * L:(i + 1) * L, 0:D]
        k = qkv[i * L:(i + 1) * L, D:2 * D]
        v = qkv[i * L:(i + 1) * L, 2 * D:3 * D]
        kp = (k[None, :, :] * msk).reshape(NH * L, D)
        s = lax.dot_general(q, kp, (((1,), (1,)), ((), ())),
                            preferred_element_type=jnp.float32)
        parts = []
        for h in range(NH):
            sh = s[:, h * L:(h + 1) * L] * scale
            m = jnp.max(sh, axis=1, keepdims=True)
            e = jnp.exp(sh - m)
            parts.append(e / jnp.sum(e, axis=1, keepdims=True))
        p = jnp.concatenate(parts, axis=1)
        vp = (v[None, :, :] * msk).reshape(NH * L, D)
        o = lax.dot_general(p, vp, (((1,), (0,)), ((), ())),
                            preferred_element_type=jnp.float32)
        outs.append(o)
    att = jnp.concatenate(outs, axis=0)
    y = att @ wot_ref[...] + ob_ref[...]
    o_ref[...] = y.reshape(nb, L, D)


def _gate_expert_kernel(x_ref, gwt_ref, gb_ref, w1_ref, b1_ref, w2_ref,
                        b2_ref, lng_ref, lnb_ref, o_ref):
    x = x_ref[...]
    logits = x @ gwt_ref[...] + gb_ref[...]
    lm = jnp.max(logits, axis=1, keepdims=True)
    z = jnp.exp(logits - lm)
    probs = z / jnp.sum(z, axis=1, keepdims=True)
    ids = lax.broadcasted_iota(jnp.int32, (TCH, NE), 1)
    m0 = jnp.max(probs, axis=1, keepdims=True)
    e0 = jnp.min(jnp.where(probs == m0, ids, NE), axis=1, keepdims=True)
    top0 = ids == e0
    p2 = jnp.where(top0, -1.0, probs)
    m1 = jnp.max(p2, axis=1, keepdims=True)
    e1 = jnp.min(jnp.where(p2 == m1, ids, NE), axis=1, keepdims=True)
    top1 = ids == e1
    denom = m0 + m1
    w = (jnp.where(top0, m0, 0.0) + jnp.where(top1, m1, 0.0)) / denom

    acc = jnp.zeros((TCH, D), jnp.float32)
    for e in range(NE):
        h = x @ w1_ref[e] + b1_ref[e].reshape(1, HID)
        h = jnp.maximum(h, 0.0)
        eo = h @ w2_ref[e] + b2_ref[e].reshape(1, D)
        acc = acc + eo * w[:, e:e + 1]
    mu = jnp.mean(acc, axis=1, keepdims=True)
    ac = acc - mu
    var = jnp.mean(ac * ac, axis=1, keepdims=True)
    y = ac * lax.rsqrt(var + 1e-5) * lng_ref[...] + lnb_ref[...]
    o_ref[...] = y * (1.0 / float(L))


def _vec_kernel(x_ref, wt_ref, b_ref, o_ref):
    o_ref[...] = x_ref[...] @ wt_ref[...] + b_ref[...]


def _head_kernel(sv_ref, cwt_ref, cb_ref, g_ref, c_ref):
    i = pl.program_id(0)

    @pl.when(i == 0)
    def _():
        g_ref[...] = jnp.zeros_like(g_ref)

    g_ref[...] += jnp.sum(sv_ref[...].reshape(TCH // B, B, D), axis=0) * (
        1.0 / float(L))

    @pl.when(i == pl.num_programs(0) - 1)
    def _():
        c_ref[...] = g_ref[...] @ cwt_ref[...] + cb_ref[...]


def _moe_layer(xn, p):
    att = pl.pallas_call(
        functools.partial(_attn_kernel, nb=NB),
        grid=(NPATCH // NB,),
        in_specs=[
            pl.BlockSpec((NB, L, D), lambda i: (i, 0, 0)),
            pl.BlockSpec((D, D), lambda i: (0, 0)),
            pl.BlockSpec((1, D), lambda i: (0, 0)),
            pl.BlockSpec((D, 3 * D), lambda i: (0, 0)),
            pl.BlockSpec((1, 3 * D), lambda i: (0, 0)),
            pl.BlockSpec((D, D), lambda i: (0, 0)),
            pl.BlockSpec((1, D), lambda i: (0, 0)),
        ],
        out_specs=pl.BlockSpec((NB, L, D), lambda i: (i, 0, 0)),
        out_shape=jax.ShapeDtypeStruct((NPATCH, B, D), jnp.float32),
    )(xn, p['ip_W'].T, p['ip_b'].reshape(1, D), p['qkv_W'].T,
      p['qkv_b'].reshape(1, 3 * D), p['o_W'].T, p['o_b'].reshape(1, D))

    xt = att.reshape(T, D)
    y = pl.pallas_call(
        _gate_expert_kernel,
        grid=(T // TCH,),
        in_specs=[
            pl.BlockSpec((TCH, D), lambda i: (i, 0)),
            pl.BlockSpec((D, NE), lambda i: (0, 0)),
            pl.BlockSpec((1, NE), lambda i: (0, 0)),
            pl.BlockSpec((NE, D, HID), lambda i: (0, 0, 0)),
            pl.BlockSpec((NE, HID), lambda i: (0, 0)),
            pl.BlockSpec((NE, HID, D), lambda i: (0, 0, 0)),
            pl.BlockSpec((NE, D), lambda i: (0, 0)),
            pl.BlockSpec((1, D), lambda i: (0, 0)),
            pl.BlockSpec((1, D), lambda i: (0, 0)),
        ],
        out_specs=pl.BlockSpec((TCH, D), lambda i: (i, 0)),
        out_shape=jax.ShapeDtypeStruct((T, D), jnp.float32),
    )(xt, p['gate_W'].T, p['gate_b'].reshape(1, NE), p['e_W1'], p['e_b1'],
      p['e_W2'], p['e_b2'], p['ln_g'].reshape(1, D), p['ln_b'].reshape(1, D))
    return y


def _vec_proj(yt, vec_Wt, vec_b):
    return pl.pallas_call(
        _vec_kernel,
        grid=(T // TCH,),
        in_specs=[
            pl.BlockSpec((TCH, D), lambda i: (i, 0)),
            pl.BlockSpec((D, D), lambda i: (0, 0)),
            pl.BlockSpec((1, D), lambda i: (0, 0)),
        ],
        out_specs=pl.BlockSpec((TCH, D), lambda i: (i, 0)),
        out_shape=jax.ShapeDtypeStruct((T, D), jnp.float32),
    )(yt, vec_Wt, vec_b)


def kernel(x, params):
    b = x.shape[0]
    xp = x.reshape(b, IMG // PS, PS, IMG // PS, PS).transpose(0, 1, 3, 2, 4)
    xp = xp.reshape(b, NPATCH, PD).transpose(1, 0, 2)

    x0 = pl.pallas_call(
        _embed_kernel,
        grid=(NPATCH // NB,),
        in_specs=[
            pl.BlockSpec((NB, B, PD), lambda i: (i, 0, 0)),
            pl.BlockSpec((PD, D), lambda i: (0, 0)),
            pl.BlockSpec((1, D), lambda i: (0, 0)),
            pl.BlockSpec((NB, D), lambda i: (i, 0)),
        ],
        out_specs=pl.BlockSpec((NB, B, D), lambda i: (i, 0, 0)),
        out_shape=jax.ShapeDtypeStruct((NPATCH, B, D), jnp.float32),
    )(xp, params['pe_W'].T, params['pe_b'].reshape(1, D),
      (params['pos_emb'].reshape(NPATCH, D)))

    vec_Wt = params['vec_W'].T
    vec_b = params['vec_b'].reshape(1, D)

    y1 = _moe_layer(x0, params['moe1'])
    fv = _vec_proj(y1, vec_Wt, vec_b)

    y2 = _moe_layer(fv.reshape(NPATCH, B, D), params['moe2'])
    sv = _vec_proj(y2, vec_Wt, vec_b)

    gv, cv = pl.pallas_call(
        _head_kernel,
        grid=(T // TCH,),
        in_specs=[
            pl.BlockSpec((TCH, D), lambda i: (i, 0)),
            pl.BlockSpec((D, D), lambda i: (0, 0)),
            pl.BlockSpec((1, D), lambda i: (0, 0)),
        ],
        out_specs=[
            pl.BlockSpec((B, D), lambda i: (0, 0)),
            pl.BlockSpec((B, D), lambda i: (0, 0)),
        ],
        out_shape=[
            jax.ShapeDtypeStruct((B, D), jnp.float32),
            jax.ShapeDtypeStruct((B, D), jnp.float32),
        ],
    )(sv, params['cls_W'].T, params['cls_b'].reshape(1, D))

    first_vector = fv.reshape(NPATCH, B, D).transpose(1, 0, 2)
    second_vector = sv.reshape(NPATCH, B, D).transpose(1, 0, 2)
    return (first_vector, second_vector, gv, cv)
